# Initial kernel scaffold; baseline (speedup 1.0000x reference)
import functools

import jax
import jax.numpy as jnp
from jax import lax
from jax.experimental import pallas as pl
from jax.experimental.pallas import tpu as pltpu
from jax.experimental.pallas import tpu_sc as plsc

B = 16384
L = 50
D = 32
NC = 2
NS = 16
NW = NC * NS
SPW = B // NW
C = 64
CL = C * L
G = 128
NG = CL // G
NCHUNK = SPW // C
IDX_ROWS_PER_CHUNK = CL // G
SG = 8


def _pool_body(idx_u_hbm, idx_m_hbm, ut_hbm, mt_hbm, out_hbm,
               idxu_v, idxm_v, rows_v, pooled_v, sem):
    wid = lax.axis_index("s") * NC + lax.axis_index("c")
    idx_row_base = wid * (SPW * L // G)
    sample_base = wid * SPW

    def chunk_body(i, carry):
        row0 = idx_row_base + i * IDX_ROWS_PER_CHUNK
        s0 = sample_base + i * C
        pltpu.sync_copy(idx_u_hbm.at[pl.ds(row0, IDX_ROWS_PER_CHUNK)], idxu_v)
        pltpu.sync_copy(idx_m_hbm.at[pl.ds(row0, IDX_ROWS_PER_CHUNK)], idxm_v)

        for tbl, idx_v, col0 in ((ut_hbm, idxu_v, 0), (mt_hbm, idxm_v, D)):
            handles = []
            for j in range(NG):
                handles.append(pltpu.async_copy(
                    tbl.at[idx_v.at[j]],
                    rows_v.at[pl.ds(j * G, G)], sem))
            for h in handles:
                h.wait()

            for g in range(C // SG):
                def red_body(l, accs):
                    out = []
                    for k in range(SG):
                        r = (g * SG + k) * L + l
                        a0 = accs[2 * k] + rows_v[r, pl.ds(0, 16)]
                        a1 = accs[2 * k + 1] + rows_v[r, pl.ds(16, 16)]
                        out.append(a0)
                        out.append(a1)
                    return tuple(out)

                zero = jnp.zeros((16,), jnp.float32)
                accs = lax.fori_loop(0, L, red_body, (zero,) * (2 * SG))
                for k in range(SG):
                    pooled_v[g * SG + k, pl.ds(col0, 16)] = accs[2 * k]
                    pooled_v[g * SG + k, pl.ds(col0 + 16, 16)] = accs[2 * k + 1]

        pltpu.sync_copy(pooled_v, out_hbm.at[pl.ds(s0, C)])
        return carry

    lax.fori_loop(0, NCHUNK, chunk_body, jnp.int32(0))


_pooler = functools.partial(
    pl.kernel,
    out_type=jax.ShapeDtypeStruct((B, 2 * D), jnp.float32),
    mesh=plsc.VectorSubcoreMesh(core_axis_name="c", subcore_axis_name="s",
                                num_cores=NC, num_subcores=NS),
    scratch_types=[
        pltpu.VMEM((IDX_ROWS_PER_CHUNK, G), jnp.int32),
        pltpu.VMEM((IDX_ROWS_PER_CHUNK, G), jnp.int32),
        pltpu.VMEM((CL, D), jnp.float32),
        pltpu.VMEM((C, 2 * D), jnp.float32),
        pltpu.SemaphoreType.DMA,
    ],
)(_pool_body)


def _mlp_body(x_ref, w1_ref, b1_ref, w2_ref, b2_ref, w3_ref, b3_ref, o_ref):
    hi = jax.lax.Precision.HIGHEST
    x = x_ref[...] * jnp.float32(1.0 / L)
    h = jnp.dot(x, w1_ref[...], preferred_element_type=jnp.float32, precision=hi)
    h = jnp.maximum(h + b1_ref[...], 0.0)
    h = jnp.dot(h, w2_ref[...], preferred_element_type=jnp.float32, precision=hi)
    h = jnp.maximum(h + b2_ref[...], 0.0)
    o_ref[...] = jnp.dot(h, w3_ref[...], preferred_element_type=jnp.float32,
                         precision=hi) + b3_ref[...]


MLP_BLK = 2048


def _mlp(pooled, W1, b1, W2, b2, W3, b3):
    grid = (B // MLP_BLK,)
    return pl.pallas_call(
        _mlp_body,
        grid=grid,
        in_specs=[
            pl.BlockSpec((MLP_BLK, 2 * D), lambda i: (i, 0)),
            pl.BlockSpec((2 * D, 256), lambda i: (0, 0)),
            pl.BlockSpec((1, 256), lambda i: (0, 0)),
            pl.BlockSpec((256, 64), lambda i: (0, 0)),
            pl.BlockSpec((1, 64), lambda i: (0, 0)),
            pl.BlockSpec((64, 1), lambda i: (0, 0)),
            pl.BlockSpec((1, 1), lambda i: (0, 0)),
        ],
        out_specs=pl.BlockSpec((MLP_BLK, 1), lambda i: (i, 0)),
        out_shape=jax.ShapeDtypeStruct((B, 1), jnp.float32),
    )(pooled, W1, b1.reshape(1, 256), W2, b2.reshape(1, 64),
      W3, b3.reshape(1, 1))


def kernel(kriteria_mentor_user, kriteria_mentor, user_table, mentor_table,
           W1, b1, W2, b2, W3, b3):
    idx_u = kriteria_mentor_user.astype(jnp.int32).reshape(B * L // G, G)
    idx_m = kriteria_mentor.astype(jnp.int32).reshape(B * L // G, G)
    pooled = _pooler(idx_u, idx_m, user_table, mentor_table)
    return _mlp(pooled, W1, b1, W2, b2, W3, b3)


# speedup vs baseline: 1.3805x; 1.3805x over previous
import functools

import jax
import jax.numpy as jnp
from jax import lax
from jax.experimental import pallas as pl
from jax.experimental.pallas import tpu as pltpu
from jax.experimental.pallas import tpu_sc as plsc

B = 16384
L = 50
D = 32
NC = 2
NS = 16
NW = NC * NS
SPW = B // NW
C = 16
LP = 56
CL = C * LP
NCHUNK = SPW // C
SG = 8


def _fire(tbl, idx_v, buf, sem):
    for s in range(C):
        pltpu.async_copy(tbl.at[idx_v.at[s, pl.ds(0, LP)]],
                         buf.at[pl.ds(s * LP, LP)], sem)


def _drain(dummy_hbm, buf, sem):
    pltpu.make_async_copy(dummy_hbm.at[pl.ds(0, CL)], buf, sem).wait()


def _reduce(buf, pooled_v, col0):
    for g in range(C // SG):
        def red_body(l, accs):
            out = []
            for k in range(SG):
                r = (g * SG + k) * LP + l
                out.append(accs[2 * k] + buf[r, pl.ds(0, 16)])
                out.append(accs[2 * k + 1] + buf[r, pl.ds(16, 16)])
            return tuple(out)

        zero = jnp.zeros((16,), jnp.float32)
        accs = lax.fori_loop(0, L, red_body, (zero,) * (2 * SG))
        for k in range(SG):
            pooled_v[g * SG + k, pl.ds(col0, 16)] = accs[2 * k]
            pooled_v[g * SG + k, pl.ds(col0 + 16, 16)] = accs[2 * k + 1]


def _pool_body(idx_u_hbm, idx_m_hbm, ut_hbm, mt_hbm, out_hbm,
               idxu_v, idxm_v, buf0, buf1, pooled_v, sem0, sem1):
    wid = lax.axis_index("s") * NC + lax.axis_index("c")
    sample_base = wid * SPW

    pltpu.sync_copy(idx_u_hbm.at[pl.ds(sample_base, C)], idxu_v)
    _fire(ut_hbm, idxu_v, buf0, sem0)

    def chunk_body(i, carry):
        s0 = sample_base + i * C
        pltpu.sync_copy(idx_m_hbm.at[pl.ds(s0, C)], idxm_v)
        _fire(mt_hbm, idxm_v, buf1, sem1)

        _drain(ut_hbm, buf0, sem0)
        _reduce(buf0, pooled_v, 0)

        ns0 = sample_base + jnp.minimum(i + 1, NCHUNK - 1) * C
        pltpu.sync_copy(idx_u_hbm.at[pl.ds(ns0, C)], idxu_v)
        _fire(ut_hbm, idxu_v, buf0, sem0)

        _drain(mt_hbm, buf1, sem1)
        _reduce(buf1, pooled_v, D)

        pltpu.sync_copy(pooled_v, out_hbm.at[pl.ds(s0, C)])
        return carry

    lax.fori_loop(0, NCHUNK, chunk_body, jnp.int32(0))
    _drain(ut_hbm, buf0, sem0)


_pooler = functools.partial(
    pl.kernel,
    out_type=jax.ShapeDtypeStruct((B, 2 * D), jnp.float32),
    mesh=plsc.VectorSubcoreMesh(core_axis_name="c", subcore_axis_name="s",
                                num_cores=NC, num_subcores=NS),
    compiler_params=pltpu.CompilerParams(use_tc_tiling_on_sc=False),
    scratch_types=[
        pltpu.VMEM((C, LP), jnp.int32),
        pltpu.VMEM((C, LP), jnp.int32),
        pltpu.VMEM((CL, D), jnp.float32),
        pltpu.VMEM((CL, D), jnp.float32),
        pltpu.VMEM((C, 2 * D), jnp.float32),
        pltpu.SemaphoreType.DMA,
        pltpu.SemaphoreType.DMA,
    ],
)(_pool_body)


def _mlp_body(x_ref, w1_ref, b1_ref, w2_ref, b2_ref, w3_ref, b3_ref, o_ref):
    hi = jax.lax.Precision.HIGHEST
    x = x_ref[...] * jnp.float32(1.0 / L)
    h = jnp.dot(x, w1_ref[...], preferred_element_type=jnp.float32, precision=hi)
    h = jnp.maximum(h + b1_ref[...], 0.0)
    h = jnp.dot(h, w2_ref[...], preferred_element_type=jnp.float32, precision=hi)
    h = jnp.maximum(h + b2_ref[...], 0.0)
    o_ref[...] = jnp.dot(h, w3_ref[...], preferred_element_type=jnp.float32,
                         precision=hi) + b3_ref[...]


MLP_BLK = 2048


def _mlp(pooled, W1, b1, W2, b2, W3, b3):
    grid = (B // MLP_BLK,)
    return pl.pallas_call(
        _mlp_body,
        grid=grid,
        in_specs=[
            pl.BlockSpec((MLP_BLK, 2 * D), lambda i: (i, 0)),
            pl.BlockSpec((2 * D, 256), lambda i: (0, 0)),
            pl.BlockSpec((1, 256), lambda i: (0, 0)),
            pl.BlockSpec((256, 64), lambda i: (0, 0)),
            pl.BlockSpec((1, 64), lambda i: (0, 0)),
            pl.BlockSpec((64, 1), lambda i: (0, 0)),
            pl.BlockSpec((1, 1), lambda i: (0, 0)),
        ],
        out_specs=pl.BlockSpec((MLP_BLK, 1), lambda i: (i, 0)),
        out_shape=jax.ShapeDtypeStruct((B, 1), jnp.float32),
    )(pooled, W1, b1.reshape(1, 256), W2, b2.reshape(1, 64),
      W3, b3.reshape(1, 1))


def kernel(kriteria_mentor_user, kriteria_mentor, user_table, mentor_table,
           W1, b1, W2, b2, W3, b3):
    pad = ((0, 0), (0, LP - L))
    idx_u = jnp.pad(kriteria_mentor_user.astype(jnp.int32), pad)
    idx_m = jnp.pad(kriteria_mentor.astype(jnp.int32), pad)
    pooled = _pooler(idx_u, idx_m, user_table, mentor_table)
    return _mlp(pooled, W1, b1, W2, b2, W3, b3)


# speedup vs baseline: 2.6864x; 1.9459x over previous
import functools

import jax
import jax.numpy as jnp
from jax import lax
from jax.experimental import pallas as pl
from jax.experimental.pallas import tpu as pltpu
from jax.experimental.pallas import tpu_sc as plsc

B = 16384
L = 50
D = 32
NC = 2
NS = 16
NW = NC * NS
SPW = B // NW
C = 32
CL = C * L
NCHUNK = SPW // C
SG = 8
IB = 128
NFULL = CL // IB
REM = CL - NFULL * IB


def _fire(tbl, idx_v, buf, sem):
    for j in range(NFULL):
        pltpu.async_copy(tbl.at[idx_v.at[pl.ds(j * IB, IB)]],
                         buf.at[pl.ds(j * IB, IB)], sem)
    pltpu.async_copy(tbl.at[idx_v.at[pl.ds(NFULL * IB, REM)]],
                     buf.at[pl.ds(NFULL * IB, REM)], sem)


def _drain(dummy_hbm, buf, sem):
    pltpu.make_async_copy(dummy_hbm.at[pl.ds(0, CL)], buf, sem).wait()


def _reduce(buf, pooled_v, col0):
    for g in range(C // SG):
        def red_body(l, accs):
            out = []
            for k in range(SG):
                r = (g * SG + k) * L + l
                out.append(accs[2 * k] + buf[r, pl.ds(0, 16)])
                out.append(accs[2 * k + 1] + buf[r, pl.ds(16, 16)])
            return tuple(out)

        zero = jnp.zeros((16,), jnp.float32)
        accs = lax.fori_loop(0, L, red_body, (zero,) * (2 * SG))
        for k in range(SG):
            pooled_v[g * SG + k, pl.ds(col0, 16)] = accs[2 * k]
            pooled_v[g * SG + k, pl.ds(col0 + 16, 16)] = accs[2 * k + 1]


def _pool_body(idx_u_hbm, idx_m_hbm, ut_hbm, mt_hbm, out_hbm,
               idxu_v, idxm_v, buf0, buf1, pooled_v, sem0, sem1):
    wid = lax.axis_index("s") * NC + lax.axis_index("c")
    sample_base = wid * SPW
    flat_base = wid * SPW * L

    pltpu.sync_copy(idx_u_hbm.at[pl.ds(flat_base, CL)], idxu_v)
    _fire(ut_hbm, idxu_v, buf0, sem0)

    def chunk_body(i, carry):
        r0 = flat_base + i * CL
        pltpu.sync_copy(idx_m_hbm.at[pl.ds(r0, CL)], idxm_v)
        _fire(mt_hbm, idxm_v, buf1, sem1)

        _drain(ut_hbm, buf0, sem0)
        _reduce(buf0, pooled_v, 0)

        nr0 = flat_base + jnp.minimum(i + 1, NCHUNK - 1) * CL
        pltpu.sync_copy(idx_u_hbm.at[pl.ds(nr0, CL)], idxu_v)
        _fire(ut_hbm, idxu_v, buf0, sem0)

        _drain(mt_hbm, buf1, sem1)
        _reduce(buf1, pooled_v, D)

        pltpu.sync_copy(pooled_v, out_hbm.at[pl.ds(sample_base + i * C, C)])
        return carry

    lax.fori_loop(0, NCHUNK, chunk_body, jnp.int32(0))
    _drain(ut_hbm, buf0, sem0)


_pooler = functools.partial(
    pl.kernel,
    out_type=jax.ShapeDtypeStruct((B, 2 * D), jnp.float32),
    mesh=plsc.VectorSubcoreMesh(core_axis_name="c", subcore_axis_name="s",
                                num_cores=NC, num_subcores=NS),
    compiler_params=pltpu.CompilerParams(use_tc_tiling_on_sc=False),
    scratch_types=[
        pltpu.VMEM((CL,), jnp.int32),
        pltpu.VMEM((CL,), jnp.int32),
        pltpu.VMEM((CL, D), jnp.float32),
        pltpu.VMEM((CL, D), jnp.float32),
        pltpu.VMEM((C, 2 * D), jnp.float32),
        pltpu.SemaphoreType.DMA,
        pltpu.SemaphoreType.DMA,
    ],
)(_pool_body)


def _mlp_body(x_ref, w1_ref, b1_ref, w2_ref, b2_ref, w3_ref, b3_ref, o_ref):
    hi = jax.lax.Precision.HIGHEST
    x = x_ref[...] * jnp.float32(1.0 / L)
    h = jnp.dot(x, w1_ref[...], preferred_element_type=jnp.float32, precision=hi)
    h = jnp.maximum(h + b1_ref[...], 0.0)
    h = jnp.dot(h, w2_ref[...], preferred_element_type=jnp.float32, precision=hi)
    h = jnp.maximum(h + b2_ref[...], 0.0)
    o_ref[...] = jnp.dot(h, w3_ref[...], preferred_element_type=jnp.float32,
                         precision=hi) + b3_ref[...]


MLP_BLK = 2048


def _mlp(pooled, W1, b1, W2, b2, W3, b3):
    grid = (B // MLP_BLK,)
    return pl.pallas_call(
        _mlp_body,
        grid=grid,
        in_specs=[
            pl.BlockSpec((MLP_BLK, 2 * D), lambda i: (i, 0)),
            pl.BlockSpec((2 * D, 256), lambda i: (0, 0)),
            pl.BlockSpec((1, 256), lambda i: (0, 0)),
            pl.BlockSpec((256, 64), lambda i: (0, 0)),
            pl.BlockSpec((1, 64), lambda i: (0, 0)),
            pl.BlockSpec((64, 1), lambda i: (0, 0)),
            pl.BlockSpec((1, 1), lambda i: (0, 0)),
        ],
        out_specs=pl.BlockSpec((MLP_BLK, 1), lambda i: (i, 0)),
        out_shape=jax.ShapeDtypeStruct((B, 1), jnp.float32),
    )(pooled, W1, b1.reshape(1, 256), W2, b2.reshape(1, 64),
      W3, b3.reshape(1, 1))


def kernel(kriteria_mentor_user, kriteria_mentor, user_table, mentor_table,
           W1, b1, W2, b2, W3, b3):
    idx_u = kriteria_mentor_user.astype(jnp.int32).reshape(B * L)
    idx_m = kriteria_mentor.astype(jnp.int32).reshape(B * L)
    pooled = _pooler(idx_u, idx_m, user_table, mentor_table)
    return _mlp(pooled, W1, b1, W2, b2, W3, b3)
